# baseline (speedup 1.0000x reference)
import functools

import jax
import jax.numpy as jnp
from jax import lax
from jax.experimental import pallas as pl
from jax.experimental.pallas import tpu as pltpu
from jax.experimental.pallas import tpu_sc as plsc

B, N, E = 8, 2048, 65536
D_STATE, D_ANN, D_IN, D_GCN = 100, 3, 103, 100
DP = 112
FC1, FC2, FC3, ACT = 128, 128, 64, 16
FLOAT_MIN = -3.4028235e38

NC, NS = 2, 16
BPC = B // NC
EPT = E // NS
CH = 128
NCH = EPT // CH
RPT = N // NS


def _sc_mesh():
    return plsc.VectorSubcoreMesh(core_axis_name="c", subcore_axis_name="s",
                                  num_cores=NC, num_subcores=NS)


@functools.partial(
    pl.kernel,
    out_type=jax.ShapeDtypeStruct((B * N,), jnp.float32),
    mesh=_sc_mesh(),
    scratch_types=[
        pltpu.VMEM((CH,), jnp.float32),
        pltpu.VMEM((NCH, CH), jnp.int32),
        pltpu.VMEM((RPT,), jnp.float32),
        pltpu.VMEM_SHARED((N,), jnp.float32),
        pltpu.SemaphoreType.DMA,
    ],
)
def _deg_kernel(dst_hbm, ones_hbm, zeros_hbm, deg_hbm, ones_v, idxd_v, zero_v, deg_sh, sem):
    c = lax.axis_index("c")
    s = lax.axis_index("s")
    pltpu.sync_copy(ones_hbm, ones_v)
    pltpu.sync_copy(zeros_hbm, zero_v)
    for bl in range(BPC):
        b = c * BPC + bl
        w = b * NS + s
        pltpu.sync_copy(zero_v, deg_sh.at[pl.ds(s * RPT, RPT)])
        pltpu.sync_copy(dst_hbm.at[w], idxd_v)
        plsc.subcore_barrier()
        descs = [pltpu.async_copy(ones_v, deg_sh.at[idxd_v.at[i]], sem, add=True)
                 for i in range(NCH)]
        for d in descs:
            d.wait()
        plsc.subcore_barrier()
        pltpu.sync_copy(deg_sh.at[pl.ds(s * RPT, RPT)],
                        deg_hbm.at[pl.ds(b * N + s * RPT, RPT)])


NSLOT = 4


@functools.partial(
    pl.kernel,
    out_type=jax.ShapeDtypeStruct((B * N, DP), jnp.float32),
    mesh=_sc_mesh(),
    scratch_types=[
        pltpu.VMEM((NCH, CH), jnp.int32),
        pltpu.VMEM((NCH, CH), jnp.int32),
        [pltpu.VMEM((CH, DP), jnp.float32)] * NSLOT,
        pltpu.VMEM((RPT, DP), jnp.float32),
        pltpu.VMEM((RPT,), jnp.int32),
        pltpu.VMEM_SHARED((N, DP), jnp.float32),
        [pltpu.SemaphoreType.DMA] * NSLOT,
        pltpu.SemaphoreType.DMA,
    ],
    compiler_params=pltpu.CompilerParams(use_tc_tiling_on_sc=False),
)
def _msg_kernel(srcg_hbm, dst_hbm, y_hbm, iota_hbm, acc_hbm,
                idxs_v, idxd_v, rows, init_v, iota_v, acc_sh, gsem, isem):
    c = lax.axis_index("c")
    s = lax.axis_index("s")
    pltpu.sync_copy(iota_hbm.at[pl.ds(s * RPT, RPT)], iota_v)

    def gstart(i, k):
        pltpu.async_copy(y_hbm.at[idxs_v.at[i]], rows[k], gsem[k])

    def gwait(k):
        pltpu.make_async_copy(y_hbm.at[idxs_v.at[0]], rows[k], gsem[k]).wait()

    for bl in range(BPC):
        b = c * BPC + bl
        w = b * NS + s
        base = b * N + s * RPT
        pltpu.sync_copy(y_hbm.at[pl.ds(base, RPT)], init_v)
        pltpu.sync_copy(init_v, acc_sh.at[pl.ds(s * RPT, RPT)])
        pltpu.async_copy(init_v, acc_sh.at[iota_v], isem, add=True)
        pltpu.sync_copy(srcg_hbm.at[w], idxs_v)
        pltpu.sync_copy(dst_hbm.at[w], idxd_v)
        pltpu.make_async_copy(init_v, acc_sh.at[iota_v], isem).wait()
        plsc.subcore_barrier()
        for k in range(NSLOT):
            gstart(k, k)

        @pl.loop(0, NCH, step=NSLOT)
        def _round(i):
            for k in range(NSLOT):
                gwait(k)
                pltpu.sync_copy(rows[k], acc_sh.at[idxd_v.at[i + k]], add=True)

                @pl.when(i + k + NSLOT < NCH)
                def _():
                    gstart(i + k + NSLOT, k)

        plsc.subcore_barrier()
        pltpu.sync_copy(acc_sh.at[pl.ds(s * RPT, RPT)],
                        acc_hbm.at[pl.ds(b * N + s * RPT, RPT)])


def _tca_body(state_ref, ann_ref, deg_ref, wgs_ref, wga_ref, y_ref):
    x = state_ref[0]
    a = ann_ref[0]
    xw = (jnp.dot(x, wgs_ref[...], preferred_element_type=jnp.float32)
          + jnp.dot(a, wga_ref[...], preferred_element_type=jnp.float32))
    deg = deg_ref[0, 0] + 2.0
    dis = lax.rsqrt(deg)
    y_ref[0] = xw * dis[:, None]


def _tcb_body(acc_ref, deg_ref, sp_ref, am_ref, cm_ref,
              bg_ref, w1_ref, b1_ref, w2s_ref, w2r_ref, b2_ref,
              w3_ref, b3_ref, w4_ref, b4_ref, out_ref):
    deg = deg_ref[0, 0] + 2.0
    dis = lax.rsqrt(deg)
    node = jnp.tanh(acc_ref[0] * dis[:, None] + bg_ref[...])
    h = jnp.maximum(
        jnp.dot(node, w1_ref[...], preferred_element_type=jnp.float32) + b1_ref[...], 0.0)
    sp = sp_ref[0, 0]
    h = jnp.maximum(
        jnp.dot(h, w2r_ref[...], preferred_element_type=jnp.float32)
        + sp[:, None] * w2s_ref[...] + b2_ref[...], 0.0)
    h = jnp.maximum(
        jnp.dot(h, w3_ref[...], preferred_element_type=jnp.float32) + b3_ref[...], 0.0)
    h = jnp.dot(h, w4_ref[...], preferred_element_type=jnp.float32) + b4_ref[...]
    mask = cm_ref[0] * am_ref[0, 0][:, None]
    out_ref[0] = h + jnp.maximum(jnp.log(mask), FLOAT_MIN)


def _full(shape):
    return pl.BlockSpec(shape, lambda b: (0,) * len(shape))


def _batched(shape):
    return pl.BlockSpec(shape, lambda b: (b,) + (0,) * (len(shape) - 1))


def kernel(state, annotations, edge_index, spill_weights, action_mask, colour_mask,
           W_gcn, b_gcn, W1, b1, W2, b2, W3, b3, W4, b4):
    f32 = jnp.float32
    src = edge_index[:, :, 0].astype(jnp.int32)
    dst = edge_index[:, :, 1].astype(jnp.int32)
    srcg = (src + (jnp.arange(B, dtype=jnp.int32) * N)[:, None]).reshape(B * NS, NCH, CH)
    dstf = dst.reshape(B * NS, NCH, CH)

    ones_c = jnp.ones((CH,), f32)
    zeros_r = jnp.zeros((RPT,), f32)
    iota_n = jnp.arange(N, dtype=jnp.int32)

    deg = _deg_kernel(dstf, ones_c, zeros_r)
    deg3 = deg.reshape(B, 1, N)

    pad = DP - D_GCN
    wgs = jnp.pad(W_gcn[:D_STATE], ((0, 0), (0, pad)))
    wga = jnp.pad(W_gcn[D_STATE:], ((0, 0), (0, pad)))

    y = pl.pallas_call(
        _tca_body,
        grid=(B,),
        in_specs=[
            _batched((1, N, D_STATE)),
            _batched((1, N, D_ANN)),
            _batched((1, 1, N)),
            _full((D_STATE, DP)),
            _full((D_ANN, DP)),
        ],
        out_specs=_batched((1, N, DP)),
        out_shape=jax.ShapeDtypeStruct((B, N, DP), f32),
    )(state, annotations, deg3, wgs, wga)

    acc = _msg_kernel(srcg, dstf, y.reshape(B * N, DP), iota_n)

    bg = jnp.pad(b_gcn, (0, pad)).reshape(1, DP)
    w1p = jnp.pad(W1, ((0, pad), (0, 0)))
    w2s = W2[0:1]
    w2r = W2[1:]
    sp3 = spill_weights.reshape(B, 1, N)
    am3 = action_mask.reshape(B, 1, N)

    out = pl.pallas_call(
        _tcb_body,
        grid=(B,),
        in_specs=[
            _batched((1, N, DP)),
            _batched((1, 1, N)),
            _batched((1, 1, N)),
            _batched((1, 1, N)),
            _batched((1, N, ACT)),
            _full((1, DP)),
            _full((DP, FC1)), _full((1, FC1)),
            _full((1, FC2)), _full((FC1, FC2)), _full((1, FC2)),
            _full((FC2, FC3)), _full((1, FC3)),
            _full((FC3, ACT)), _full((1, ACT)),
        ],
        out_specs=_batched((1, N, ACT)),
        out_shape=jax.ShapeDtypeStruct((B, N, ACT), f32),
    )(acc.reshape(B, N, DP), deg3, sp3, am3, colour_mask,
      bg, w1p, b1.reshape(1, FC1), w2s, w2r, b2.reshape(1, FC2),
      W3, b3.reshape(1, FC3), W4, b4.reshape(1, ACT))

    return out.reshape(B, N * ACT)


# speedup vs baseline: 1.0819x; 1.0819x over previous
import functools

import jax
import jax.numpy as jnp
from jax import lax
from jax.experimental import pallas as pl
from jax.experimental.pallas import tpu as pltpu
from jax.experimental.pallas import tpu_sc as plsc

B, N, E = 8, 2048, 65536
D_STATE, D_ANN, D_IN, D_GCN = 100, 3, 103, 100
DP = 128
FC1, FC2, FC3, ACT = 128, 128, 64, 16
FLOAT_MIN = -3.4028235e38

NC, NS = 2, 16
BPC = B // NC
EPT = E // NS
CH = 128
NCH = EPT // CH
RPT = N // NS


def _sc_mesh():
    return plsc.VectorSubcoreMesh(core_axis_name="c", subcore_axis_name="s",
                                  num_cores=NC, num_subcores=NS)


@functools.partial(
    pl.kernel,
    out_type=jax.ShapeDtypeStruct((B * N,), jnp.float32),
    mesh=_sc_mesh(),
    scratch_types=[
        pltpu.VMEM((CH,), jnp.float32),
        pltpu.VMEM((NCH, CH), jnp.int32),
        pltpu.VMEM((RPT,), jnp.float32),
        pltpu.VMEM_SHARED((N,), jnp.float32),
        pltpu.SemaphoreType.DMA,
    ],
)
def _deg_kernel(dst_hbm, ones_hbm, zeros_hbm, deg_hbm, ones_v, idxd_v, zero_v, deg_sh, sem):
    c = lax.axis_index("c")
    s = lax.axis_index("s")
    pltpu.sync_copy(ones_hbm, ones_v)
    pltpu.sync_copy(zeros_hbm, zero_v)
    for bl in range(BPC):
        b = c * BPC + bl
        w = b * NS + s
        pltpu.sync_copy(zero_v, deg_sh.at[pl.ds(s * RPT, RPT)])
        pltpu.sync_copy(dst_hbm.at[w], idxd_v)
        plsc.subcore_barrier()
        descs = [pltpu.async_copy(ones_v, deg_sh.at[idxd_v.at[i]], sem, add=True)
                 for i in range(NCH)]
        for d in descs:
            d.wait()
        plsc.subcore_barrier()
        pltpu.sync_copy(deg_sh.at[pl.ds(s * RPT, RPT)],
                        deg_hbm.at[pl.ds(b * N + s * RPT, RPT)])


NSLOT = 4
NCHT = BPC * NCH
BRES = 2
NCHR = BRES * NCH


@functools.partial(
    pl.kernel,
    out_type=jax.ShapeDtypeStruct((B * N, DP), jnp.float32),
    mesh=_sc_mesh(),
    scratch_types=[
        pltpu.VMEM((NCHR, CH), jnp.int32),
        pltpu.VMEM((NCHR, CH), jnp.int32),
        [pltpu.VMEM((CH, DP), jnp.float32)] * NSLOT,
        pltpu.VMEM_SHARED((BRES * N, DP), jnp.float32),
        [pltpu.SemaphoreType.DMA] * NSLOT,
    ],
)
def _msg_kernel(srcg_hbm, dstl_hbm, y_hbm, acc_hbm,
                idxs_v, idxd_v, rows, acc_sh, gsem):
    c = lax.axis_index("c")
    s = lax.axis_index("s")
    w = c * NS + s

    def gstart(i, k):
        pltpu.async_copy(y_hbm.at[idxs_v.at[i]], rows[k], gsem[k])

    def gwait(k):
        pltpu.make_async_copy(y_hbm.at[idxs_v.at[0]], rows[k], gsem[k]).wait()

    for half in range(BPC // BRES):
        for bl in range(BRES):
            b = c * BPC + half * BRES + bl
            pltpu.sync_copy(y_hbm.at[pl.ds(b * N + s * RPT, RPT)],
                            acc_sh.at[pl.ds(bl * N + s * RPT, RPT)])
        pltpu.sync_copy(srcg_hbm.at[w, pl.ds(half * NCHR, NCHR)], idxs_v)
        pltpu.sync_copy(dstl_hbm.at[w, pl.ds(half * NCHR, NCHR)], idxd_v)
        plsc.subcore_barrier()

        for k in range(NSLOT):
            gstart(k, k)

        @pl.loop(0, NCHR, step=NSLOT)
        def _round(i):
            for k in range(NSLOT):
                gwait(k)
                pltpu.sync_copy(rows[k], acc_sh.at[idxd_v.at[i + k]], add=True)

                @pl.when(i + k + NSLOT < NCHR)
                def _():
                    gstart(i + k + NSLOT, k)

        plsc.subcore_barrier()
        for bl in range(BRES):
            b = c * BPC + half * BRES + bl
            pltpu.sync_copy(acc_sh.at[pl.ds(bl * N + s * RPT, RPT)],
                            acc_hbm.at[pl.ds(b * N + s * RPT, RPT)])


def _tca_body(state_ref, ann_ref, deg_ref, wgs_ref, wga_ref, y_ref):
    x = state_ref[0]
    a = ann_ref[0]
    xw = (jnp.dot(x, wgs_ref[...], preferred_element_type=jnp.float32)
          + jnp.dot(a, wga_ref[...], preferred_element_type=jnp.float32))
    deg = deg_ref[0, 0] + 2.0
    dis = lax.rsqrt(deg)
    y_ref[0] = xw * dis[:, None]


def _tcb_body(acc_ref, y_ref, deg_ref, sp_ref, am_ref, cm_ref,
              bg_ref, w1_ref, b1_ref, w2s_ref, w2r_ref, b2_ref,
              w3_ref, b3_ref, w4_ref, b4_ref, out_ref):
    deg = deg_ref[0, 0] + 2.0
    dis = lax.rsqrt(deg)
    node = jnp.tanh((acc_ref[0] + y_ref[0]) * dis[:, None] + bg_ref[...])
    h = jnp.maximum(
        jnp.dot(node, w1_ref[...], preferred_element_type=jnp.float32) + b1_ref[...], 0.0)
    sp = sp_ref[0, 0]
    h = jnp.maximum(
        jnp.dot(h, w2r_ref[...], preferred_element_type=jnp.float32)
        + sp[:, None] * w2s_ref[...] + b2_ref[...], 0.0)
    h = jnp.maximum(
        jnp.dot(h, w3_ref[...], preferred_element_type=jnp.float32) + b3_ref[...], 0.0)
    h = jnp.dot(h, w4_ref[...], preferred_element_type=jnp.float32) + b4_ref[...]
    mask = cm_ref[0] * am_ref[0, 0][:, None]
    out_ref[0] = h + jnp.maximum(jnp.log(mask), FLOAT_MIN)


def _full(shape):
    return pl.BlockSpec(shape, lambda b: (0,) * len(shape))


def _batched(shape):
    return pl.BlockSpec(shape, lambda b: (b,) + (0,) * (len(shape) - 1))


def kernel(state, annotations, edge_index, spill_weights, action_mask, colour_mask,
           W_gcn, b_gcn, W1, b1, W2, b2, W3, b3, W4, b4):
    f32 = jnp.float32
    src = edge_index[:, :, 0].astype(jnp.int32)
    dst = edge_index[:, :, 1].astype(jnp.int32)
    dstf = dst.reshape(B * NS, NCH, CH)

    def _tile_major(a):
        return (a.reshape(NC, BPC, NS, NCH, CH)
                 .transpose(0, 2, 1, 3, 4)
                 .reshape(NC * NS, NCHT, CH))

    srcg = _tile_major(src + (jnp.arange(B, dtype=jnp.int32) * N)[:, None])
    dstl = _tile_major(dst + ((jnp.arange(B, dtype=jnp.int32) % BRES) * N)[:, None])

    ones_c = jnp.ones((CH,), f32)
    zeros_r = jnp.zeros((RPT,), f32)

    deg = _deg_kernel(dstf, ones_c, zeros_r)
    deg3 = deg.reshape(B, 1, N)

    pad = DP - D_GCN
    wgs = jnp.pad(W_gcn[:D_STATE], ((0, 0), (0, pad)))
    wga = jnp.pad(W_gcn[D_STATE:], ((0, 0), (0, pad)))

    y = pl.pallas_call(
        _tca_body,
        grid=(B,),
        in_specs=[
            _batched((1, N, D_STATE)),
            _batched((1, N, D_ANN)),
            _batched((1, 1, N)),
            _full((D_STATE, DP)),
            _full((D_ANN, DP)),
        ],
        out_specs=_batched((1, N, DP)),
        out_shape=jax.ShapeDtypeStruct((B, N, DP), f32),
    )(state, annotations, deg3, wgs, wga)

    acc = _msg_kernel(srcg, dstl, y.reshape(B * N, DP))

    bg = jnp.pad(b_gcn, (0, pad)).reshape(1, DP)
    w1p = jnp.pad(W1, ((0, pad), (0, 0)))
    w2s = W2[0:1]
    w2r = W2[1:]
    sp3 = spill_weights.reshape(B, 1, N)
    am3 = action_mask.reshape(B, 1, N)

    out = pl.pallas_call(
        _tcb_body,
        grid=(B,),
        in_specs=[
            _batched((1, N, DP)),
            _batched((1, N, DP)),
            _batched((1, 1, N)),
            _batched((1, 1, N)),
            _batched((1, 1, N)),
            _batched((1, N, ACT)),
            _full((1, DP)),
            _full((DP, FC1)), _full((1, FC1)),
            _full((1, FC2)), _full((FC1, FC2)), _full((1, FC2)),
            _full((FC2, FC3)), _full((1, FC3)),
            _full((FC3, ACT)), _full((1, ACT)),
        ],
        out_specs=_batched((1, N, ACT)),
        out_shape=jax.ShapeDtypeStruct((B, N, ACT), f32),
    )(acc.reshape(B, N, DP), y, deg3, sp3, am3, colour_mask,
      bg, w1p, b1.reshape(1, FC1), w2s, w2r, b2.reshape(1, FC2),
      W3, b3.reshape(1, FC3), W4, b4.reshape(1, ACT))

    return out.reshape(B, N * ACT)


# speedup vs baseline: 1.0871x; 1.0048x over previous
import functools

import jax
import jax.numpy as jnp
from jax import lax
from jax.experimental import pallas as pl
from jax.experimental.pallas import tpu as pltpu
from jax.experimental.pallas import tpu_sc as plsc

B, N, E = 8, 2048, 65536
D_STATE, D_ANN, D_IN, D_GCN = 100, 3, 103, 100
DP = 128
FC1, FC2, FC3, ACT = 128, 128, 64, 16
FLOAT_MIN = -3.4028235e38

NC, NS = 2, 16
BPC = B // NC
EPT = E // NS
CH = 128
NCH = EPT // CH
RPT = N // NS


def _sc_mesh():
    return plsc.VectorSubcoreMesh(core_axis_name="c", subcore_axis_name="s",
                                  num_cores=NC, num_subcores=NS)


@functools.partial(
    pl.kernel,
    out_type=jax.ShapeDtypeStruct((B * N,), jnp.float32),
    mesh=_sc_mesh(),
    scratch_types=[
        pltpu.VMEM((CH,), jnp.float32),
        pltpu.VMEM((BPC * NCH, CH), jnp.int32),
        pltpu.VMEM((BPC * RPT,), jnp.float32),
        pltpu.VMEM_SHARED((BPC * N,), jnp.float32),
        pltpu.SemaphoreType.DMA,
    ],
)
def _deg_kernel(dstb_hbm, ones_hbm, zeros_hbm, deg_hbm, ones_v, idxd_v, zero_v, deg_sh, sem):
    c = lax.axis_index("c")
    s = lax.axis_index("s")
    w = c * NS + s
    pltpu.sync_copy(ones_hbm, ones_v)
    pltpu.sync_copy(zeros_hbm, zero_v)
    for bl in range(BPC):
        pltpu.sync_copy(zero_v.at[pl.ds(bl * RPT, RPT)],
                        deg_sh.at[pl.ds(bl * N + s * RPT, RPT)])
    pltpu.sync_copy(dstb_hbm.at[w], idxd_v)
    plsc.subcore_barrier()
    descs = [pltpu.async_copy(ones_v, deg_sh.at[idxd_v.at[i]], sem, add=True)
             for i in range(BPC * NCH)]
    for d in descs:
        d.wait()
    plsc.subcore_barrier()
    for bl in range(BPC):
        b = c * BPC + bl
        pltpu.sync_copy(deg_sh.at[pl.ds(bl * N + s * RPT, RPT)],
                        deg_hbm.at[pl.ds(b * N + s * RPT, RPT)])


NSLOT = 4
NCHT = BPC * NCH
BRES = 2
NCHR = BRES * NCH


@functools.partial(
    pl.kernel,
    out_type=jax.ShapeDtypeStruct((B * N, DP), jnp.float32),
    mesh=_sc_mesh(),
    scratch_types=[
        pltpu.VMEM((NCHR, CH), jnp.int32),
        pltpu.VMEM((NCHR, CH), jnp.int32),
        [pltpu.VMEM((CH, DP), jnp.float32)] * NSLOT,
        pltpu.VMEM_SHARED((BRES * N, DP), jnp.float32),
        [pltpu.SemaphoreType.DMA] * NSLOT,
    ],
)
def _msg_kernel(srcg_hbm, dstl_hbm, y_hbm, acc_hbm,
                idxs_v, idxd_v, rows, acc_sh, gsem):
    c = lax.axis_index("c")
    s = lax.axis_index("s")
    w = c * NS + s

    def gstart(i, k):
        pltpu.async_copy(y_hbm.at[idxs_v.at[i]], rows[k], gsem[k])

    def gwait(k):
        pltpu.make_async_copy(y_hbm.at[idxs_v.at[0]], rows[k], gsem[k]).wait()

    for half in range(BPC // BRES):
        for bl in range(BRES):
            b = c * BPC + half * BRES + bl
            pltpu.sync_copy(y_hbm.at[pl.ds(b * N + s * RPT, RPT)],
                            acc_sh.at[pl.ds(bl * N + s * RPT, RPT)])
        pltpu.sync_copy(srcg_hbm.at[w, pl.ds(half * NCHR, NCHR)], idxs_v)
        pltpu.sync_copy(dstl_hbm.at[w, pl.ds(half * NCHR, NCHR)], idxd_v)
        plsc.subcore_barrier()

        for k in range(NSLOT):
            gstart(k, k)

        @pl.loop(0, NCHR, step=NSLOT)
        def _round(i):
            for k in range(NSLOT):
                gwait(k)
                pltpu.sync_copy(rows[k], acc_sh.at[idxd_v.at[i + k]], add=True)

                @pl.when(i + k + NSLOT < NCHR)
                def _():
                    gstart(i + k + NSLOT, k)

        plsc.subcore_barrier()
        for bl in range(BRES):
            b = c * BPC + half * BRES + bl
            pltpu.sync_copy(acc_sh.at[pl.ds(bl * N + s * RPT, RPT)],
                            acc_hbm.at[pl.ds(b * N + s * RPT, RPT)])


def _tca_body(state_ref, ann_ref, deg_ref, wgs_ref, wga_ref, y_ref):
    x = state_ref[0]
    a = ann_ref[0]
    xw = (jnp.dot(x, wgs_ref[...], preferred_element_type=jnp.float32)
          + jnp.dot(a, wga_ref[...], preferred_element_type=jnp.float32))
    deg = deg_ref[0, 0] + 2.0
    dis = lax.rsqrt(deg)
    y_ref[0] = xw * dis[:, None]


def _tcb_body(acc_ref, y_ref, deg_ref, sp_ref, am_ref, cm_ref,
              bg_ref, w1_ref, b1_ref, w2s_ref, w2r_ref, b2_ref,
              w3_ref, b3_ref, w4_ref, b4_ref, out_ref):
    deg = deg_ref[0, 0] + 2.0
    dis = lax.rsqrt(deg)
    node = jnp.tanh((acc_ref[0] + y_ref[0]) * dis[:, None] + bg_ref[...])
    h = jnp.maximum(
        jnp.dot(node, w1_ref[...], preferred_element_type=jnp.float32) + b1_ref[...], 0.0)
    sp = sp_ref[0, 0]
    h = jnp.maximum(
        jnp.dot(h, w2r_ref[...], preferred_element_type=jnp.float32)
        + sp[:, None] * w2s_ref[...] + b2_ref[...], 0.0)
    h = jnp.maximum(
        jnp.dot(h, w3_ref[...], preferred_element_type=jnp.float32) + b3_ref[...], 0.0)
    h = jnp.dot(h, w4_ref[...], preferred_element_type=jnp.float32) + b4_ref[...]
    mask = cm_ref[0] * am_ref[0, 0][:, None]
    out_ref[0] = h + jnp.maximum(jnp.log(mask), FLOAT_MIN)


def _full(shape):
    return pl.BlockSpec(shape, lambda b: (0,) * len(shape))


def _batched(shape):
    return pl.BlockSpec(shape, lambda b: (b,) + (0,) * (len(shape) - 1))


def kernel(state, annotations, edge_index, spill_weights, action_mask, colour_mask,
           W_gcn, b_gcn, W1, b1, W2, b2, W3, b3, W4, b4):
    f32 = jnp.float32
    src = edge_index[:, :, 0].astype(jnp.int32)
    dst = edge_index[:, :, 1].astype(jnp.int32)

    def _tile_major(a):
        return (a.reshape(NC, BPC, NS, NCH, CH)
                 .transpose(0, 2, 1, 3, 4)
                 .reshape(NC * NS, NCHT, CH))

    srcg = _tile_major(src + (jnp.arange(B, dtype=jnp.int32) * N)[:, None])
    dstl = _tile_major(dst + ((jnp.arange(B, dtype=jnp.int32) % BRES) * N)[:, None])
    dstb = _tile_major(dst + ((jnp.arange(B, dtype=jnp.int32) % BPC) * N)[:, None])

    ones_c = jnp.ones((CH,), f32)
    zeros_r = jnp.zeros((BPC * RPT,), f32)

    deg = _deg_kernel(dstb, ones_c, zeros_r)
    deg3 = deg.reshape(B, 1, N)

    pad = DP - D_GCN
    wgs = jnp.pad(W_gcn[:D_STATE], ((0, 0), (0, pad)))
    wga = jnp.pad(W_gcn[D_STATE:], ((0, 0), (0, pad)))

    y = pl.pallas_call(
        _tca_body,
        grid=(B,),
        in_specs=[
            _batched((1, N, D_STATE)),
            _batched((1, N, D_ANN)),
            _batched((1, 1, N)),
            _full((D_STATE, DP)),
            _full((D_ANN, DP)),
        ],
        out_specs=_batched((1, N, DP)),
        out_shape=jax.ShapeDtypeStruct((B, N, DP), f32),
    )(state, annotations, deg3, wgs, wga)

    acc = _msg_kernel(srcg, dstl, y.reshape(B * N, DP))

    bg = jnp.pad(b_gcn, (0, pad)).reshape(1, DP)
    w1p = jnp.pad(W1, ((0, pad), (0, 0)))
    w2s = W2[0:1]
    w2r = W2[1:]
    sp3 = spill_weights.reshape(B, 1, N)
    am3 = action_mask.reshape(B, 1, N)

    out = pl.pallas_call(
        _tcb_body,
        grid=(B,),
        in_specs=[
            _batched((1, N, DP)),
            _batched((1, N, DP)),
            _batched((1, 1, N)),
            _batched((1, 1, N)),
            _batched((1, 1, N)),
            _batched((1, N, ACT)),
            _full((1, DP)),
            _full((DP, FC1)), _full((1, FC1)),
            _full((1, FC2)), _full((FC1, FC2)), _full((1, FC2)),
            _full((FC2, FC3)), _full((1, FC3)),
            _full((FC3, ACT)), _full((1, ACT)),
        ],
        out_specs=_batched((1, N, ACT)),
        out_shape=jax.ShapeDtypeStruct((B, N, ACT), f32),
    )(acc.reshape(B, N, DP), y, deg3, sp3, am3, colour_mask,
      bg, w1p, b1.reshape(1, FC1), w2s, w2r, b2.reshape(1, FC2),
      W3, b3.reshape(1, FC3), W4, b4.reshape(1, ACT))

    return out.reshape(B, N * ACT)
